# Initial kernel scaffold; baseline (speedup 1.0000x reference)
import jax
import jax.numpy as jnp
from jax.experimental import pallas as pl


def kernel(x, cond, w_ada, b_ada, w_proj, b_proj):
    raise NotImplementedError("write your pallas kernel here")


# speedup vs baseline: 1.5745x; 1.5745x over previous
import jax
import jax.numpy as jnp
from jax.experimental import pallas as pl
from jax.experimental.pallas import tpu as pltpu

_EPS = 1e-6


def _mod_kernel(cond_ref, w_ada_ref, b_ada_ref, mod_ref):
    c = cond_ref[...]
    s = c * jax.nn.sigmoid(c)
    mod_ref[...] = (
        jnp.dot(s, w_ada_ref[...], preferred_element_type=jnp.float32)
        + b_ada_ref[...]
    )


def _main_kernel(x_ref, mod_ref, w_proj_ref, b_proj_ref, out_ref):
    d = x_ref.shape[-1]
    x = x_ref[0]
    mu = jnp.mean(x, axis=-1, keepdims=True)
    xc = x - mu
    var = jnp.mean(xc * xc, axis=-1, keepdims=True)
    xn = xc * jax.lax.rsqrt(var + _EPS)
    scale = mod_ref[0, :, :d]
    shift = mod_ref[0, :, d:]
    y = xn * (1.0 + scale) + shift
    y = y * jax.nn.sigmoid(y)
    out_ref[0] = (
        jnp.dot(y, w_proj_ref[...], preferred_element_type=jnp.float32)
        + b_proj_ref[...]
    )


def kernel(x, cond, w_ada, b_ada, w_proj, b_proj, interpret=False):
    B, T, D = x.shape
    OUT = w_proj.shape[1]
    TBLK = 1024

    mod = pl.pallas_call(
        _mod_kernel,
        out_shape=jax.ShapeDtypeStruct((B, 2 * D), jnp.float32),
        interpret=interpret,
    )(cond, w_ada, b_ada.reshape(1, 2 * D))

    mod3 = mod.reshape(B, 1, 2 * D)
    grid = (B, T // TBLK)
    out = pl.pallas_call(
        _main_kernel,
        out_shape=jax.ShapeDtypeStruct((B, T, OUT), jnp.float32),
        grid=grid,
        in_specs=[
            pl.BlockSpec((1, TBLK, D), lambda b, t: (b, t, 0)),
            pl.BlockSpec((1, 1, 2 * D), lambda b, t: (b, 0, 0)),
            pl.BlockSpec((D, OUT), lambda b, t: (0, 0)),
            pl.BlockSpec((1, OUT), lambda b, t: (0, 0)),
        ],
        out_specs=pl.BlockSpec((1, TBLK, OUT), lambda b, t: (b, t, 0)),
        compiler_params=pltpu.CompilerParams(
            dimension_semantics=("parallel", "arbitrary"),
            vmem_limit_bytes=48 * 1024 * 1024,
        ),
        interpret=interpret,
    )(x, mod3, w_proj, b_proj.reshape(1, OUT))
    return out
